# Initial kernel scaffold; baseline (speedup 1.0000x reference)
import jax
import jax.numpy as jnp
from jax.experimental import pallas as pl


def kernel(P1, X1, P2, S2, Wz, bz, Wr, br, Ws, bs, Wfc, bfc):
    raise NotImplementedError("write your pallas kernel here")


# speedup vs baseline: 7.6295x; 7.6295x over previous
import functools

import jax
import jax.numpy as jnp
from jax import lax
from jax.experimental import pallas as pl
from jax.experimental.pallas import tpu as pltpu
from jax.experimental.pallas import tpu_sc as plsc

B = 4
N = 2048
K = 16
CI = 128
CO = 256
C3 = 3 * CO
NBLK_A = 256
NBLK_C = 512
SLABS = C3 // K
NWORKERS = 32
SLABS_PER_W = SLABS * B // NWORKERS


def _stage_a_body(p1_ref, p2t_ref, p2_ref, x1_ref, s2_ref,
                  wgs_ref, wgd_ref, wax_ref, wad_ref, ba_ref,
                  idx_ref, g_ref, a_ref):
    p1 = p1_ref[0]
    p2t = p2t_ref[0]

    dx = p1[:, 0:1] - p2t[0:1, :]
    dy = p1[:, 1:2] - p2t[1:2, :]
    dz = p1[:, 2:3] - p2t[2:3, :]
    d = dx * dx + dy * dy + dz * dz

    col = lax.broadcasted_iota(jnp.int32, d.shape, 1)
    vals = d
    cols = []
    for _ in range(K):
        m = jnp.min(vals, axis=1, keepdims=True)
        cand = jnp.where(vals == m, col, N * 2)
        arg = jnp.min(cand, axis=1, keepdims=True)
        cols.append(arg)
        vals = jnp.where(col == arg, jnp.float32(jnp.inf), vals)
    b = pl.program_id(0)
    idx_ref[0] = jnp.concatenate(cols, axis=1) + b * N

    hi = jax.lax.Precision.HIGHEST
    s2 = s2_ref[0]
    p2 = p2_ref[0]
    g = lax.dot_general(s2, wgs_ref[...], (((0,), (1,)), ((), ())),
                        precision=hi)
    g = g + lax.dot_general(p2, wgd_ref[...], (((1,), (1,)), ((), ())),
                            precision=hi)
    g_ref[0] = g

    x1 = x1_ref[0]
    a = lax.dot_general(x1, wax_ref[...], (((0,), (1,)), ((), ())),
                        precision=hi)
    a = a + lax.dot_general(p1, wad_ref[...], (((1,), (1,)), ((), ())),
                            precision=hi)
    a_ref[0] = a + ba_ref[...]


PPW = (B * N) // NWORKERS
CHUNK = 64


def _gather_max_body(g_hbm, idx_hbm, out_hbm, idx_v, rows_v, o_v, sem):
    cid = lax.axis_index("c")
    sid = lax.axis_index("s")
    wid = sid * 2 + cid
    base = wid * PPW
    pltpu.sync_copy(idx_hbm.at[pl.ds(base, PPW)], idx_v)

    for ci in range(PPW // CHUNK):
        def body(p, carry):
            n = ci * CHUNK + p
            pltpu.make_async_copy(g_hbm.at[idx_v.at[n]], rows_v, sem).start()
            pltpu.make_async_copy(g_hbm.at[idx_v.at[n]], rows_v, sem).wait()
            for j in range(C3 // K):
                sl = pl.ds(j * K, K)
                acc = rows_v[0, sl]
                for k in range(1, K):
                    acc = jnp.maximum(acc, rows_v[k, sl])
                o_v[p, sl] = acc
            return carry

        lax.fori_loop(0, CHUNK, body, 0)
        pltpu.sync_copy(o_v, out_hbm.at[pl.ds(base + ci * CHUNK, CHUNK)])


def _stage_c_body(pre_ref, a_ref, x1_ref, wfc_ref, bfc_ref, out_ref):
    hi = jax.lax.Precision.HIGHEST
    t = pre_ref[0] + a_ref[0]
    zn = 1.0 / (1.0 + jnp.exp(-t[:, :CO]))
    rn = 1.0 / (1.0 + jnp.exp(-t[:, CO:2 * CO]))
    sold_n = t[:, 2 * CO:]
    rs = rn * sold_n
    x1 = x1_ref[0]
    snew = lax.dot_general(wfc_ref[:, :CI], x1, (((1,), (0,)), ((), ())),
                           precision=hi)
    snew = snew + lax.dot_general(wfc_ref[:, CI:], rs,
                                  (((1,), (1,)), ((), ())), precision=hi)
    snew = jnp.tanh(snew + bfc_ref[...])
    zc = zn.T
    soldc = sold_n.T
    out_ref[0] = zc * soldc + (1.0 - zc) * snew


def kernel(P1, X1, P2, S2, Wz, bz, Wr, br, Ws, bs, Wfc, bfc):
    f32 = jnp.float32
    WGs = jnp.concatenate([Wz[:, :CO], Wr[:, :CO], Ws[:, :CO]], 0)
    WGd = jnp.concatenate([Wz[:, CO + CI:], Wr[:, CO + CI:], Ws[:, CO:]], 0)
    WAx = jnp.concatenate([Wz[:, CO:CO + CI], Wr[:, CO:CO + CI],
                           jnp.zeros((CO, CI), f32)], 0)
    WAd = -WGd
    BA = jnp.concatenate([bz, br, bs]).reshape(1, C3)
    P2T = jnp.transpose(P2, (0, 2, 1))

    grid_a = (B, N // NBLK_A)
    idx_g, G, A = pl.pallas_call(
        _stage_a_body,
        grid=grid_a,
        in_specs=[
            pl.BlockSpec((1, NBLK_A, 3), lambda b, i: (b, i, 0)),
            pl.BlockSpec((1, 3, N), lambda b, i: (b, 0, 0)),
            pl.BlockSpec((1, NBLK_A, 3), lambda b, i: (b, i, 0)),
            pl.BlockSpec((1, CI, NBLK_A), lambda b, i: (b, 0, i)),
            pl.BlockSpec((1, CO, NBLK_A), lambda b, i: (b, 0, i)),
            pl.BlockSpec((C3, CO), lambda b, i: (0, 0)),
            pl.BlockSpec((C3, 3), lambda b, i: (0, 0)),
            pl.BlockSpec((C3, CI), lambda b, i: (0, 0)),
            pl.BlockSpec((C3, 3), lambda b, i: (0, 0)),
            pl.BlockSpec((1, C3), lambda b, i: (0, 0)),
        ],
        out_specs=[
            pl.BlockSpec((1, NBLK_A, K), lambda b, i: (b, i, 0)),
            pl.BlockSpec((1, NBLK_A, C3), lambda b, i: (b, i, 0)),
            pl.BlockSpec((1, NBLK_A, C3), lambda b, i: (b, i, 0)),
        ],
        out_shape=[
            jax.ShapeDtypeStruct((B, N, K), jnp.int32),
            jax.ShapeDtypeStruct((B, N, C3), f32),
            jax.ShapeDtypeStruct((B, N, C3), f32),
        ],
    )(P1, P2T, P2, X1, S2, WGs, WGd, WAx, WAd, BA)

    mesh = plsc.VectorSubcoreMesh(core_axis_name="c", subcore_axis_name="s")
    pre = pl.kernel(
        _gather_max_body,
        out_type=jax.ShapeDtypeStruct((B * N, C3), f32),
        mesh=mesh,
        scratch_types=[
            pltpu.VMEM((PPW, K), jnp.int32),
            pltpu.VMEM((K, C3), f32),
            pltpu.VMEM((CHUNK, C3), f32),
            pltpu.SemaphoreType.DMA,
        ],
    )(G.reshape(B * N, C3), idx_g.reshape(B * N, K))
    pre = pre.reshape(B, N, C3)

    grid_c = (B, N // NBLK_C)
    S1 = pl.pallas_call(
        _stage_c_body,
        grid=grid_c,
        in_specs=[
            pl.BlockSpec((1, NBLK_C, C3), lambda b, i: (b, i, 0)),
            pl.BlockSpec((1, NBLK_C, C3), lambda b, i: (b, i, 0)),
            pl.BlockSpec((1, CI, NBLK_C), lambda b, i: (b, 0, i)),
            pl.BlockSpec((CO, CO + CI), lambda b, i: (0, 0)),
            pl.BlockSpec((CO, 1), lambda b, i: (0, 0)),
        ],
        out_specs=pl.BlockSpec((1, CO, NBLK_C), lambda b, i: (b, 0, i)),
        out_shape=jax.ShapeDtypeStruct((B, CO, N), f32),
    )(pre, A, X1, Wfc, bfc.reshape(CO, 1))

    return (P1, S1)
